# Initial kernel scaffold; baseline (speedup 1.0000x reference)
import jax
import jax.numpy as jnp
from jax.experimental import pallas as pl


def kernel(xyz, points, W1, b1, W2, b2):
    raise NotImplementedError("write your pallas kernel here")


# speedup vs baseline: 6.6860x; 6.6860x over previous
import numpy as np
import jax
import jax.numpy as jnp
from jax.experimental import pallas as pl
from jax.experimental.pallas import tpu as pltpu

_NPOINT = 128
_NSAMPLE = 32
_R2 = np.float32(0.4 * 0.4)
_NEG = np.float32(-3.0e38)
_BIG = np.int32(2**30)
_CH = 128


def _body(xyz8_ref, xyzr_ref, f16_ref, w1_ref, b1_ref, w2_ref, b2_ref,
          out_ref, a_ref):
    n = xyz8_ref.shape[2]
    xr = xyzr_ref[0, 0]
    yr = xyzr_ref[0, 1]
    zr = xyzr_ref[0, 2]
    rows, cols = xr.shape

    flat_iota = (jax.lax.broadcasted_iota(jnp.int32, (rows, cols), 0) * cols
                 + jax.lax.broadcasted_iota(jnp.int32, (rows, cols), 1))

    lane_row = jax.lax.broadcasted_iota(jnp.int32, (1, _NPOINT), 1)
    lane_col = jax.lax.broadcasted_iota(jnp.int32, (_NPOINT, 1), 0)

    def pick(sel, arr):
        return jnp.max(jnp.where(sel, arr, _NEG))

    sel0 = flat_iota == 0
    cx0 = pick(sel0, xr)
    cy0 = pick(sel0, yr)
    cz0 = pick(sel0, zr)
    zrow = jnp.zeros((1, _NPOINT), jnp.float32)
    zcol = jnp.zeros((_NPOINT, 1), jnp.float32)

    def place(i, val, row, col):
        return (jnp.where(lane_row == i, val, row),
                jnp.where(lane_col == i, val, col))

    cxr, cxc = place(0, cx0, zrow, zcol)
    cyr, cyc = place(0, cy0, zrow, zcol)
    czr, czc = place(0, cz0, zrow, zcol)

    def fps_step(i, carry):
        cx, cy, cz, dists, cxr, cyr, czr, cxc, cyc, czc = carry
        d = (xr - cx) * (xr - cx) + (yr - cy) * (yr - cy)
        d = d + (zr - cz) * (zr - cz)
        dists = jnp.minimum(dists, d)
        m = jnp.max(dists)
        sel = dists == m
        nxt = jnp.min(jnp.where(sel, flat_iota, _BIG))
        seln = flat_iota == nxt
        ncx = pick(seln, xr)
        ncy = pick(seln, yr)
        ncz = pick(seln, zr)
        cxr, cxc = place(i, ncx, cxr, cxc)
        cyr, cyc = place(i, ncy, cyr, cyc)
        czr, czc = place(i, ncz, czr, czc)
        return ncx, ncy, ncz, dists, cxr, cyr, czr, cxc, cyc, czc

    dists0 = jnp.full((rows, cols), 1e10, dtype=jnp.float32)
    (_, _, _, _, cxr, cyr, czr, cxs, cys, czs) = jax.lax.fori_loop(
        1, _NPOINT, fps_step,
        (cx0, cy0, cz0, dists0, cxr, cyr, czr, cxc, cyc, czc))

    a_ref[...] = jnp.dot(w1_ref[...], f16_ref[0],
                         preferred_element_type=jnp.float32)

    ctr3 = jnp.concatenate([cxr, cyr, czr], axis=0)

    nch = n // _CH
    tri = (jax.lax.broadcasted_iota(jnp.int32, (_CH, _CH), 0)
           <= jax.lax.broadcasted_iota(jnp.int32, (_CH, _CH), 1)
           ).astype(jnp.float32)
    nsf = jnp.float32(_NSAMPLE)

    def chunk_cond(carry):
        k, cnt, _ = carry
        return jnp.logical_and(k < nch, jnp.min(cnt) < nsf)

    def chunk_body(carry):
        k, cnt, acc = carry
        base = pl.multiple_of(k * _CH, _CH)
        xc = xyz8_ref[0, :, pl.ds(base, _CH)]
        px, py, pz = xc[0:1], xc[1:2], xc[2:3]
        dx = cxs - px
        dy = cys - py
        dz = czs - pz
        d2 = dx * dx + dy * dy
        d2 = d2 + dz * dz
        mask = d2 < _R2
        local = jnp.dot(mask.astype(jnp.float32), tri,
                        preferred_element_type=jnp.float32)
        sel = jnp.logical_and(mask, (cnt + local) <= nsf)
        ac = a_ref[:, pl.ds(base, _CH)]
        m3 = jnp.where(sel[None, :, :], ac[:, None, :], _NEG)
        acc = jnp.maximum(acc, jnp.max(m3, axis=2))
        cnt = cnt + local[:, _CH - 1:_CH]
        return k + jnp.int32(1), cnt, acc

    acc0 = jnp.full((64, _NPOINT), _NEG, jnp.float32)
    cnt0 = jnp.zeros((_NPOINT, 1), jnp.float32)
    _, _, acc = jax.lax.while_loop(chunk_cond, chunk_body,
                                   (jnp.int32(0), cnt0, acc0))

    w1x = w1_ref[:, 0:3]
    t = b1_ref[...] - jnp.dot(w1x, ctr3,
                              preferred_element_type=jnp.float32)
    l1 = jnp.maximum(acc + t, 0.0)
    g2 = jnp.concatenate([ctr3, l1], axis=0)
    h2 = jnp.dot(w2_ref[...], g2,
                 preferred_element_type=jnp.float32) + b2_ref[...]
    h2 = jnp.maximum(h2, 0.0)
    out_ref[0] = jnp.max(h2, axis=1, keepdims=True)


def kernel(xyz, points, W1, b1, W2, b2):
    B, N, _ = xyz.shape
    OUT = W1.shape[0]
    xyzT = jnp.transpose(xyz, (0, 2, 1))
    xyz8 = jnp.concatenate(
        [xyzT, jnp.zeros((B, 5, N), xyz.dtype)], axis=1)
    xyzr = xyzT.reshape(B, 3, N // 128, 128)
    f16 = jnp.concatenate(
        [xyzT, points,
         jnp.zeros((B, 16 - 3 - points.shape[1], N), xyz.dtype)], axis=1)
    W1p = jnp.concatenate(
        [W1, jnp.zeros((OUT, 16 - W1.shape[1]), W1.dtype)], axis=1)
    b1c = b1.reshape(OUT, 1)
    b2c = b2.reshape(OUT, 1)

    out = pl.pallas_call(
        _body,
        grid=(B,),
        in_specs=[
            pl.BlockSpec((1, 8, N), lambda b: (b, 0, 0)),
            pl.BlockSpec((1, 3, N // 128, 128), lambda b: (b, 0, 0, 0)),
            pl.BlockSpec((1, 16, N), lambda b: (b, 0, 0)),
            pl.BlockSpec((OUT, 16), lambda b: (0, 0)),
            pl.BlockSpec((OUT, 1), lambda b: (0, 0)),
            pl.BlockSpec((OUT, OUT + 3), lambda b: (0, 0)),
            pl.BlockSpec((OUT, 1), lambda b: (0, 0)),
        ],
        out_specs=pl.BlockSpec((1, OUT, 1), lambda b: (b, 0, 0)),
        out_shape=jax.ShapeDtypeStruct((B, OUT, 1), jnp.float32),
        scratch_shapes=[
            pltpu.VMEM((OUT, N), jnp.float32),
        ],
        compiler_params=pltpu.CompilerParams(
            dimension_semantics=("arbitrary",)),
    )(xyz8, xyzr, f16, W1p, b1c, W2, b2c)
    return out.reshape(B, OUT)
